# baseline (speedup 1.0000x reference)
import functools

import jax
import jax.numpy as jnp
from jax.experimental import pallas as pl
from jax.experimental.pallas import tpu as pltpu

K = 32
BLOCK_T = 512


def _body(x_ref, we_ref, wd_ref, b_ref, xhat_ref, xhid_ref):
    xb = x_ref[...] - b_ref[...]
    h = jax.lax.dot_general(
        xb, we_ref[...], (((1,), (1,)), ((), ())),
        preferred_element_type=jnp.float32)
    m = jax.lax.bitcast_convert_type(h, jnp.int32)
    m = m ^ jnp.where(m < 0, jnp.int32(0x7FFFFFFF), jnp.int32(0))
    t16 = (m >> 16).astype(jnp.int16)
    l16 = ((m & 0xFFFF) - 32768).astype(jnp.int16)
    one = jnp.bfloat16(1)
    zero = jnp.bfloat16(0)

    lo = jnp.full((t16.shape[0], 1), -32768, jnp.int32)
    hi = jnp.full((t16.shape[0], 1), 32767, jnp.int32)
    for _ in range(16):
        mid = (lo + hi + 1) >> 1
        mid16 = mid.astype(jnp.int16)
        cnt = jnp.sum(jnp.where(t16 >= mid16, one, zero), axis=1,
                      keepdims=True).astype(jnp.int32)
        ge = cnt >= K
        lo = jnp.where(ge, mid, lo)
        hi = jnp.where(ge, hi, mid - 1)
    tb16 = lo.astype(jnp.int16)
    c_hi = jnp.sum(jnp.where(t16 > tb16, one, zero), axis=1,
                   keepdims=True).astype(jnp.int32)
    r = K - c_hi

    q = jnp.where(t16 == tb16, l16, jnp.int16(-32768))
    lo2 = jnp.full_like(lo, -32768)
    hi2 = jnp.full_like(lo, 32767)
    for _ in range(16):
        mid = (lo2 + hi2 + 1) >> 1
        mid16 = mid.astype(jnp.int16)
        cnt = jnp.sum(jnp.where(q >= mid16, one, zero), axis=1,
                      keepdims=True).astype(jnp.int32)
        ge = cnt >= r
        lo2 = jnp.where(ge, mid, lo2)
        hi2 = jnp.where(ge, hi2, mid - 1)
    ql16 = lo2.astype(jnp.int16)
    keep = (t16 > tb16) | ((t16 == tb16) & (l16 >= ql16))
    xhid = jnp.where(keep, h, 0.0)
    xhid_ref[...] = xhid
    xhat_ref[...] = jax.lax.dot_general(
        xhid, wd_ref[...], (((1,), (1,)), ((), ())),
        preferred_element_type=jnp.float32) + b_ref[...]


@jax.jit
def kernel(x, w_enc, w_dec, b_dec):
    b, s, d_model = x.shape
    d_feat = w_enc.shape[0]
    n_tok = b * s
    xf = x.reshape(n_tok, d_model)
    b2 = b_dec.reshape(1, d_model)
    grid = (n_tok // BLOCK_T,)
    xhat, xhid = pl.pallas_call(
        _body,
        grid=grid,
        in_specs=[
            pl.BlockSpec((BLOCK_T, d_model), lambda i: (i, 0)),
            pl.BlockSpec((d_feat, d_model), lambda i: (0, 0)),
            pl.BlockSpec((d_model, d_feat), lambda i: (0, 0)),
            pl.BlockSpec((1, d_model), lambda i: (0, 0)),
        ],
        out_specs=[
            pl.BlockSpec((BLOCK_T, d_model), lambda i: (i, 0)),
            pl.BlockSpec((BLOCK_T, d_feat), lambda i: (i, 0)),
        ],
        out_shape=[
            jax.ShapeDtypeStruct((n_tok, d_model), jnp.float32),
            jax.ShapeDtypeStruct((n_tok, d_feat), jnp.float32),
        ],
        compiler_params=pltpu.CompilerParams(
            dimension_semantics=("arbitrary",),
        ),
    )(xf, w_enc, w_dec, b2)
    return (xhat.reshape(b, s, d_model), xhid.reshape(b, s, d_feat))
